# baseline (speedup 1.0000x reference)
import jax
import jax.numpy as jnp
from jax.experimental import pallas as pl
from jax.experimental.pallas import tpu as pltpu

_B, _H, _S, _D = 8, 16, 2048, 128
_L = 16


def _body(pos_ref, kc, vc, kv, vv, ko, vo, sem_k, sem_v, sem_u):
    copies = []
    for b in range(_B):
        ck = pltpu.make_async_copy(kc.at[b], ko.at[b], sem_k)
        cv = pltpu.make_async_copy(vc.at[b], vo.at[b], sem_v)
        ck.start()
        cv.start()
        copies.append((ck, cv))
    for ck, cv in copies:
        ck.wait()
        cv.wait()
    p0 = pos_ref[0]
    uk = pltpu.make_async_copy(kv, ko.at[:, :, pl.ds(p0, _L), :], sem_u)
    uv = pltpu.make_async_copy(vv, vo.at[:, :, pl.ds(p0, _L), :], sem_u)
    uk.start()
    uv.start()
    uk.wait()
    uv.wait()


def kernel(k_cache, v_cache, input_pos, k_val, v_val):
    any_spec = pl.BlockSpec(memory_space=pltpu.HBM)
    out = pl.pallas_call(
        _body,
        in_specs=[
            pl.BlockSpec(memory_space=pltpu.SMEM),
            any_spec, any_spec, any_spec, any_spec,
        ],
        out_specs=[any_spec, any_spec],
        out_shape=[jax.ShapeDtypeStruct((_B, _H, _S, _D), jnp.float32)] * 2,
        scratch_shapes=[pltpu.SemaphoreType.DMA] * 3,
    )(input_pos, k_cache, v_cache, k_val, v_val)
    return (out[0], out[1])


# speedup vs baseline: 43.5382x; 43.5382x over previous
import jax
import jax.numpy as jnp
from jax.experimental import pallas as pl
from jax.experimental.pallas import tpu as pltpu

_B, _H, _S, _D = 8, 16, 2048, 128
_L = 16


def _body(pos_ref, kc_ref, vc_ref, kv_ref, vv_ref, ko_ref, vo_ref):
    ko_ref[...] = kc_ref[...]
    vo_ref[...] = vc_ref[...]
    p0 = pos_ref[0]
    ko_ref[0, 0, pl.ds(p0, _L), :] = kv_ref[0, 0, :, :]
    vo_ref[0, 0, pl.ds(p0, _L), :] = vv_ref[0, 0, :, :]


def kernel(k_cache, v_cache, input_pos, k_val, v_val):
    cache_spec = pl.BlockSpec((1, 1, _S, _D), lambda i, j, pos: (i, j, 0, 0))
    val_spec = pl.BlockSpec((1, 1, _L, _D), lambda i, j, pos: (i, j, 0, 0))
    out = pl.pallas_call(
        _body,
        grid_spec=pltpu.PrefetchScalarGridSpec(
            num_scalar_prefetch=1,
            grid=(_B, _H),
            in_specs=[cache_spec, cache_spec, val_spec, val_spec],
            out_specs=[cache_spec, cache_spec],
        ),
        out_shape=[jax.ShapeDtypeStruct((_B, _H, _S, _D), jnp.float32)] * 2,
        compiler_params=pltpu.CompilerParams(
            dimension_semantics=("arbitrary", "arbitrary"),
        ),
    )(input_pos, k_cache, v_cache, k_val, v_val)
    return (out[0], out[1])


# speedup vs baseline: 48.7515x; 1.1197x over previous
import jax
import jax.numpy as jnp
from jax.experimental import pallas as pl
from jax.experimental.pallas import tpu as pltpu

_B, _H, _S, _D = 8, 16, 2048, 128
_L = 16


_HB = 4


def _body(pos_ref, kc_ref, vc_ref, kv_ref, vv_ref, ko_ref, vo_ref):
    ko_ref[...] = kc_ref[...]
    vo_ref[...] = vc_ref[...]
    p0 = pos_ref[0]
    for h in range(_HB):
        ko_ref[0, h, pl.ds(p0, _L), :] = kv_ref[0, h, :, :]
        vo_ref[0, h, pl.ds(p0, _L), :] = vv_ref[0, h, :, :]


def kernel(k_cache, v_cache, input_pos, k_val, v_val):
    cache_spec = pl.BlockSpec((1, _HB, _S, _D), lambda i, j, pos: (i, j, 0, 0))
    val_spec = pl.BlockSpec((1, _HB, _L, _D), lambda i, j, pos: (i, j, 0, 0))
    out = pl.pallas_call(
        _body,
        grid_spec=pltpu.PrefetchScalarGridSpec(
            num_scalar_prefetch=1,
            grid=(_B, _H // _HB),
            in_specs=[cache_spec, cache_spec, val_spec, val_spec],
            out_specs=[cache_spec, cache_spec],
        ),
        out_shape=[jax.ShapeDtypeStruct((_B, _H, _S, _D), jnp.float32)] * 2,
        compiler_params=pltpu.CompilerParams(
            dimension_semantics=("arbitrary", "arbitrary"),
        ),
    )(input_pos, k_cache, v_cache, k_val, v_val)
    return (out[0], out[1])


# speedup vs baseline: 48.8698x; 1.0024x over previous
import jax
import jax.numpy as jnp
from jax.experimental import pallas as pl
from jax.experimental.pallas import tpu as pltpu

_B, _H, _S, _D = 8, 16, 2048, 128
_L = 16
_HB = 8


def _body(pos_ref, c_ref, v_ref, o_ref):
    o_ref[...] = c_ref[...]
    p0 = pos_ref[0]
    for h in range(_HB):
        o_ref[0, h, pl.ds(p0, _L), :] = v_ref[0, h, :, :]


def _update(cache, pos, val):
    cache_spec = pl.BlockSpec((1, _HB, _S, _D), lambda i, j, p: (i, j, 0, 0))
    val_spec = pl.BlockSpec((1, _HB, _L, _D), lambda i, j, p: (i, j, 0, 0))
    return pl.pallas_call(
        _body,
        grid_spec=pltpu.PrefetchScalarGridSpec(
            num_scalar_prefetch=1,
            grid=(_B, _H // _HB),
            in_specs=[cache_spec, val_spec],
            out_specs=cache_spec,
        ),
        out_shape=jax.ShapeDtypeStruct((_B, _H, _S, _D), jnp.float32),
        compiler_params=pltpu.CompilerParams(
            dimension_semantics=("arbitrary", "arbitrary"),
        ),
    )(pos, cache, val)


def kernel(k_cache, v_cache, input_pos, k_val, v_val):
    return (_update(k_cache, input_pos, k_val),
            _update(v_cache, input_pos, v_val))
